# Initial kernel scaffold; baseline (speedup 1.0000x reference)
import jax
import jax.numpy as jnp
from jax.experimental import pallas as pl


def kernel(z, emb):
    raise NotImplementedError("write your pallas kernel here")


# speedup vs baseline: 2.5865x; 2.5865x over previous
import functools

import jax
import jax.numpy as jnp
from jax import lax
from jax.experimental import pallas as pl
from jax.experimental.pallas import tpu as pltpu
from jax.experimental.pallas import tpu_sc as plsc

_NE = 1024
_ED = 128
_NBIT = 10
_BETA = 0.25
_SNR_DB = 10.0

_TOK_BLK = 1024

_NC, _NS = 2, 16
_NW = _NC * _NS
_GCHUNK = 200


def _vq_argmin_body(zf_ref, embt_ref, am_ref, om_ref, ridx_ref, loss_ref):
    zf = zf_ref[...]
    embt = embt_ref[...]
    mm = jnp.dot(zf, embt, preferred_element_type=jnp.float32)
    zsq = jnp.sum(zf * zf, axis=1, keepdims=True)
    ssq = jnp.sum(embt * embt, axis=0, keepdims=True)
    d = (zsq + ssq) - 2.0 * mm
    dmin = jnp.min(d, axis=1, keepdims=True)
    ids = lax.broadcasted_iota(jnp.int32, d.shape, 1)
    idx = jnp.min(jnp.where(d == dmin, ids, _NE), axis=1, keepdims=True)
    ridx_ref[...] = (idx & am_ref[...]) | om_ref[...]

    @pl.when(pl.program_id(0) == 0)
    def _init():
        loss_ref[...] = jnp.zeros_like(loss_ref)

    loss_ref[...] += jnp.sum(dmin, axis=0, keepdims=True)


def _tc_vq(zp, embt, am, om):
    ntok = zp.shape[0]
    nblk = ntok // _TOK_BLK
    return pl.pallas_call(
        _vq_argmin_body,
        grid=(nblk,),
        in_specs=[
            pl.BlockSpec((_TOK_BLK, _ED), lambda i: (i, 0)),
            pl.BlockSpec((_ED, _NE), lambda i: (0, 0)),
            pl.BlockSpec((_TOK_BLK, 1), lambda i: (i, 0)),
            pl.BlockSpec((_TOK_BLK, 1), lambda i: (i, 0)),
        ],
        out_specs=[
            pl.BlockSpec((_TOK_BLK, 1), lambda i: (i, 0)),
            pl.BlockSpec((1, 1), lambda i: (0, 0)),
        ],
        out_shape=[
            jax.ShapeDtypeStruct((ntok, 1), jnp.int32),
            jax.ShapeDtypeStruct((1, 1), jnp.float32),
        ],
    )(zp, embt, am, om)


def _sc_gather(ridx, emb):
    ntok = ridx.shape[0]
    bpw = ntok // _NW
    nch = bpw // _GCHUNK
    mesh = plsc.VectorSubcoreMesh(core_axis_name="c", subcore_axis_name="s")

    @functools.partial(
        pl.kernel,
        mesh=mesh,
        out_type=jax.ShapeDtypeStruct((ntok, _ED), jnp.float32),
        scratch_types=[
            pltpu.VMEM((_GCHUNK,), jnp.int32),
            pltpu.VMEM((_GCHUNK, _ED), jnp.float32),
            pltpu.SemaphoreType.DMA,
        ],
    )
    def gather_k(ridx_hbm, emb_hbm, out_hbm, idx_v, row_v, sem):
        wid = lax.axis_index("s") * _NC + lax.axis_index("c")
        base = wid * bpw

        def chunk(c, carry):
            off = base + c * _GCHUNK
            pltpu.sync_copy(ridx_hbm.at[pl.ds(off, _GCHUNK)], idx_v)
            pltpu.async_copy(emb_hbm.at[idx_v], row_v, sem).wait()
            pltpu.sync_copy(row_v, out_hbm.at[pl.ds(off, _GCHUNK)])
            return carry

        lax.fori_loop(0, nch, chunk, 0)

    return gather_k(ridx, emb)


def _channel_masks(ntok):
    shifts = jnp.arange(_NBIT - 1, -1, -1, dtype=jnp.int32)
    snr_linear = 10.0 ** (_SNR_DB / 10.0)
    noise_std = jnp.sqrt(jnp.asarray(0.5 / snr_linear, dtype=jnp.float32))
    noise = jax.random.normal(jax.random.key(1234), (ntok * _NBIT,),
                              dtype=jnp.float32) * noise_std
    n = noise.reshape(-1, _NBIT)
    pw = jnp.left_shift(jnp.int32(1), shifts)
    keep1 = (1.0 + n) >= 0.0
    make1 = (-1.0 + n) >= 0.0
    and_mask = jnp.sum(jnp.where(keep1, pw, 0), axis=1).astype(jnp.int32)
    or_mask = jnp.sum(jnp.where(make1, pw, 0), axis=1).astype(jnp.int32)
    return and_mask, or_mask


def kernel(z, emb):
    b, c, h, w = z.shape
    ntok = b * h * w
    zp = jnp.transpose(z, (0, 2, 3, 1)).reshape(ntok, _ED)
    embt = jnp.transpose(emb)
    am, om = _channel_masks(ntok)
    ridx2, loss_sum = _tc_vq(zp, embt, am.reshape(ntok, 1), om.reshape(ntok, 1))
    loss = loss_sum[0, 0] * jnp.float32((1.0 + _BETA) / float(z.size))
    zq = _sc_gather(ridx2.reshape(ntok), emb)
    out = jnp.transpose(zq.reshape(b, h, w, _ED), (0, 3, 1, 2))
    return loss, out


# speedup vs baseline: 2.7184x; 1.0510x over previous
import functools

import jax
import jax.numpy as jnp
from jax import lax
from jax.experimental import pallas as pl
from jax.experimental.pallas import tpu as pltpu
from jax.experimental.pallas import tpu_sc as plsc

_NE = 1024
_ED = 128
_NBIT = 10
_BETA = 0.25
_SNR_DB = 10.0

_TOK_BLK = 1024

_NC, _NS = 2, 16
_NW = _NC * _NS
_GCHUNK = 200


def _vq_argmin_body(zf_ref, embt_ref, am_ref, om_ref, ridx_ref, loss_ref):
    zf = zf_ref[...]
    embt = embt_ref[...]
    mm = jnp.dot(zf, embt, preferred_element_type=jnp.float32)
    zsq = jnp.sum(zf * zf, axis=1, keepdims=True)
    ssq = jnp.sum(embt * embt, axis=0, keepdims=True)
    d = (zsq + ssq) - 2.0 * mm
    di = lax.bitcast_convert_type(d, jnp.int32)
    dmin_i = jnp.min(di, axis=1, keepdims=True)
    dmin = lax.bitcast_convert_type(dmin_i, jnp.float32)
    ids = lax.broadcasted_iota(jnp.int32, d.shape, 1)
    idx = jnp.min(jnp.where(di == dmin_i, ids, _NE), axis=1, keepdims=True)
    ridx_ref[...] = (idx & am_ref[...]) | om_ref[...]

    @pl.when(pl.program_id(0) == 0)
    def _init():
        loss_ref[...] = jnp.zeros_like(loss_ref)

    loss_ref[...] += jnp.sum(dmin, axis=0, keepdims=True)


def _tc_vq(zp, embt, am, om):
    ntok = zp.shape[0]
    nblk = ntok // _TOK_BLK
    return pl.pallas_call(
        _vq_argmin_body,
        grid=(nblk,),
        in_specs=[
            pl.BlockSpec((_TOK_BLK, _ED), lambda i: (i, 0)),
            pl.BlockSpec((_ED, _NE), lambda i: (0, 0)),
            pl.BlockSpec((_TOK_BLK, 1), lambda i: (i, 0)),
            pl.BlockSpec((_TOK_BLK, 1), lambda i: (i, 0)),
        ],
        out_specs=[
            pl.BlockSpec((_TOK_BLK, 1), lambda i: (i, 0)),
            pl.BlockSpec((1, 1), lambda i: (0, 0)),
        ],
        out_shape=[
            jax.ShapeDtypeStruct((ntok, 1), jnp.int32),
            jax.ShapeDtypeStruct((1, 1), jnp.float32),
        ],
    )(zp, embt, am, om)


def _sc_gather(ridx, emb):
    ntok = ridx.shape[0]
    bpw = ntok // _NW
    nch = bpw // _GCHUNK
    mesh = plsc.VectorSubcoreMesh(core_axis_name="c", subcore_axis_name="s")

    @functools.partial(
        pl.kernel,
        mesh=mesh,
        out_type=jax.ShapeDtypeStruct((ntok, _ED), jnp.float32),
        scratch_types=[
            pltpu.VMEM((_GCHUNK,), jnp.int32),
            pltpu.VMEM((_GCHUNK, _ED), jnp.float32),
            pltpu.SemaphoreType.DMA,
        ],
    )
    def gather_k(ridx_hbm, emb_hbm, out_hbm, idx_v, row_v, sem):
        wid = lax.axis_index("s") * _NC + lax.axis_index("c")
        base = wid * bpw

        def chunk(c, carry):
            off = base + c * _GCHUNK
            pltpu.sync_copy(ridx_hbm.at[pl.ds(off, _GCHUNK)], idx_v)
            pltpu.async_copy(emb_hbm.at[idx_v], row_v, sem).wait()
            pltpu.sync_copy(row_v, out_hbm.at[pl.ds(off, _GCHUNK)])
            return carry

        lax.fori_loop(0, nch, chunk, 0)

    return gather_k(ridx, emb)


@functools.lru_cache(maxsize=None)
def _channel_masks(ntok):
    cpu = jax.devices("cpu")[0]
    with jax.ensure_compile_time_eval(), jax.default_device(cpu):
        shifts = jnp.arange(_NBIT - 1, -1, -1, dtype=jnp.int32)
        snr_linear = 10.0 ** (_SNR_DB / 10.0)
        noise_std = jnp.sqrt(jnp.asarray(0.5 / snr_linear, dtype=jnp.float32))
        noise = jax.random.normal(jax.random.key(1234), (ntok * _NBIT,),
                                  dtype=jnp.float32) * noise_std
        n = noise.reshape(-1, _NBIT)
        pw = jnp.left_shift(jnp.int32(1), shifts)
        keep1 = (1.0 + n) >= 0.0
        make1 = (-1.0 + n) >= 0.0
        and_mask = jnp.sum(jnp.where(keep1, pw, 0), axis=1).astype(jnp.int32)
        or_mask = jnp.sum(jnp.where(make1, pw, 0), axis=1).astype(jnp.int32)
        import numpy as _np
        return _np.asarray(and_mask.reshape(ntok, 1)), _np.asarray(or_mask.reshape(ntok, 1))


def kernel(z, emb):
    b, c, h, w = z.shape
    ntok = b * h * w
    zp = jnp.transpose(z, (0, 2, 3, 1)).reshape(ntok, _ED)
    embt = jnp.transpose(emb)
    am, om = _channel_masks(ntok)
    ridx2, loss_sum = _tc_vq(zp, embt, jnp.asarray(am), jnp.asarray(om))
    loss = loss_sum[0, 0] * jnp.float32((1.0 + _BETA) / float(z.size))
    zq = _sc_gather(ridx2.reshape(ntok), emb)
    out = jnp.transpose(zq.reshape(b, h, w, _ED), (0, 3, 1, 2))
    return loss, out


# speedup vs baseline: 4.4708x; 1.6446x over previous
import functools

import jax
import jax.numpy as jnp
from jax import lax
from jax.experimental import pallas as pl
from jax.experimental.pallas import tpu as pltpu
from jax.experimental.pallas import tpu_sc as plsc

_NE = 1024
_ED = 128
_NBIT = 10
_BETA = 0.25
_SNR_DB = 10.0

_TOK_BLK = 1024

_NC, _NS = 2, 16
_NW = _NC * _NS
_GCHUNK = 200


def _vq_argmin_body(zf_ref, embt_ref, am_ref, om_ref, ridx_ref, loss_ref):
    zf = zf_ref[...]
    embt = embt_ref[...]
    mm = jnp.dot(zf, embt, preferred_element_type=jnp.float32)
    zsq = jnp.sum(zf * zf, axis=1, keepdims=True)
    ssq = jnp.sum(embt * embt, axis=0, keepdims=True)
    d = (zsq + ssq) - 2.0 * mm
    di = lax.bitcast_convert_type(d, jnp.int32)
    dmin_i = jnp.min(di, axis=1, keepdims=True)
    dmin = lax.bitcast_convert_type(dmin_i, jnp.float32)
    ids = lax.broadcasted_iota(jnp.int32, d.shape, 1)
    idx = jnp.min(jnp.where(di == dmin_i, ids, _NE), axis=1, keepdims=True)
    ridx_ref[...] = (idx & am_ref[...]) | om_ref[...]

    @pl.when(pl.program_id(0) == 0)
    def _init():
        loss_ref[...] = jnp.zeros_like(loss_ref)

    loss_ref[...] += jnp.sum(dmin, axis=0, keepdims=True)


def _tc_vq(zp, embt, am, om):
    ntok = zp.shape[0]
    nblk = ntok // _TOK_BLK
    return pl.pallas_call(
        _vq_argmin_body,
        grid=(nblk,),
        in_specs=[
            pl.BlockSpec((_TOK_BLK, _ED), lambda i: (i, 0)),
            pl.BlockSpec((_ED, _NE), lambda i: (0, 0)),
            pl.BlockSpec((_TOK_BLK, 1), lambda i: (i, 0)),
            pl.BlockSpec((_TOK_BLK, 1), lambda i: (i, 0)),
        ],
        out_specs=[
            pl.BlockSpec((_TOK_BLK, 1), lambda i: (i, 0)),
            pl.BlockSpec((1, 1), lambda i: (0, 0)),
        ],
        out_shape=[
            jax.ShapeDtypeStruct((ntok, 1), jnp.int32),
            jax.ShapeDtypeStruct((1, 1), jnp.float32),
        ],
    )(zp, embt, am, om)


def _sc_gather(ridx, emb):
    ntok = ridx.shape[0]
    bpw = ntok // _NW
    nch = bpw // _GCHUNK
    mesh = plsc.VectorSubcoreMesh(core_axis_name="c", subcore_axis_name="s")

    @functools.partial(
        pl.kernel,
        mesh=mesh,
        out_type=jax.ShapeDtypeStruct((ntok, _ED), jnp.float32),
        scratch_types=[
            pltpu.VMEM((_GCHUNK,), jnp.int32),
            pltpu.VMEM((_GCHUNK, _ED), jnp.float32),
            pltpu.SemaphoreType.DMA,
        ],
    )
    def gather_k(ridx_hbm, emb_hbm, out_hbm, idx_v, row_v, sem):
        wid = lax.axis_index("s") * _NC + lax.axis_index("c")
        base = wid * bpw

        def chunk(c, carry):
            off = base + c * _GCHUNK
            pltpu.sync_copy(ridx_hbm.at[pl.ds(off, _GCHUNK)], idx_v)
            pltpu.async_copy(emb_hbm.at[idx_v], row_v, sem).wait()
            pltpu.sync_copy(row_v, out_hbm.at[pl.ds(off, _GCHUNK)])
            return carry

        lax.fori_loop(0, nch, chunk, 0)

    return gather_k(ridx, emb)


@functools.lru_cache(maxsize=None)
def _channel_masks(nbatch, npos):
    ntok = nbatch * npos
    cpu = jax.devices("cpu")[0]
    with jax.ensure_compile_time_eval(), jax.default_device(cpu):
        shifts = jnp.arange(_NBIT - 1, -1, -1, dtype=jnp.int32)
        snr_linear = 10.0 ** (_SNR_DB / 10.0)
        noise_std = jnp.sqrt(jnp.asarray(0.5 / snr_linear, dtype=jnp.float32))
        noise = jax.random.normal(jax.random.key(1234), (ntok * _NBIT,),
                                  dtype=jnp.float32) * noise_std
        n = noise.reshape(-1, _NBIT)
        pw = jnp.left_shift(jnp.int32(1), shifts)
        keep1 = (1.0 + n) >= 0.0
        make1 = (-1.0 + n) >= 0.0
        and_mask = jnp.sum(jnp.where(keep1, pw, 0), axis=1).astype(jnp.int32)
        or_mask = jnp.sum(jnp.where(make1, pw, 0), axis=1).astype(jnp.int32)
        import numpy as _np
        am = _np.asarray(and_mask).reshape(nbatch, npos).T.reshape(ntok, 1)
        om = _np.asarray(or_mask).reshape(nbatch, npos).T.reshape(ntok, 1)
        return _np.ascontiguousarray(am), _np.ascontiguousarray(om)


def kernel(z, emb):
    b, c, h, w = z.shape
    hw = h * w
    ntok = b * hw
    zp = jnp.transpose(z.reshape(b, c, hw), (2, 0, 1)).reshape(ntok, _ED)
    embt = jnp.transpose(emb)
    am, om = _channel_masks(b, hw)
    ridx2, loss_sum = _tc_vq(zp, embt, jnp.asarray(am), jnp.asarray(om))
    loss = loss_sum[0, 0] * jnp.float32((1.0 + _BETA) / float(z.size))
    zq = _sc_gather(ridx2.reshape(ntok), emb)
    out = jnp.transpose(zq.reshape(hw, b, _ED), (1, 2, 0)).reshape(b, c, h, w)
    return loss, out


# speedup vs baseline: 4.6113x; 1.0314x over previous
import functools

import jax
import jax.numpy as jnp
from jax import lax
from jax.experimental import pallas as pl
from jax.experimental.pallas import tpu as pltpu
from jax.experimental.pallas import tpu_sc as plsc

_NE = 1024
_ED = 128
_NBIT = 10
_BETA = 0.25
_SNR_DB = 10.0

_TOK_BLK = 1024

_NC, _NS = 2, 16
_NW = _NC * _NS
_GCHUNK = 200


def _vq_argmin_body(zf_ref, embt_ref, am_ref, om_ref, ridx_ref, loss_ref):
    zf = zf_ref[...]
    embt = embt_ref[...]
    mm = jnp.dot(zf, embt, preferred_element_type=jnp.float32)
    zsq = jnp.sum(zf * zf, axis=1, keepdims=True)
    ssq = jnp.sum(embt * embt, axis=0, keepdims=True)
    d = (zsq + ssq) - 2.0 * mm
    di = lax.bitcast_convert_type(d, jnp.int32)
    dmin_i = jnp.min(di, axis=1, keepdims=True)
    dmin = lax.bitcast_convert_type(dmin_i, jnp.float32)
    ids = lax.broadcasted_iota(jnp.int32, d.shape, 1)
    idx = jnp.min(jnp.where(di == dmin_i, ids, _NE), axis=1, keepdims=True)
    idx_l = jnp.transpose(idx, (1, 0))
    ridx_ref[...] = ((idx_l & am_ref[0]) | om_ref[0])[None]

    @pl.when(pl.program_id(0) == 0)
    def _init():
        loss_ref[...] = jnp.zeros_like(loss_ref)

    loss_ref[...] += jnp.sum(dmin, axis=0, keepdims=True)


def _tc_vq(zp, embt, am, om):
    ntok = zp.shape[0]
    nblk = ntok // _TOK_BLK
    return pl.pallas_call(
        _vq_argmin_body,
        grid=(nblk,),
        in_specs=[
            pl.BlockSpec((_TOK_BLK, _ED), lambda i: (i, 0)),
            pl.BlockSpec((_ED, _NE), lambda i: (0, 0)),
            pl.BlockSpec((1, 1, _TOK_BLK), lambda i: (i, 0, 0)),
            pl.BlockSpec((1, 1, _TOK_BLK), lambda i: (i, 0, 0)),
        ],
        out_specs=[
            pl.BlockSpec((1, 1, _TOK_BLK), lambda i: (i, 0, 0)),
            pl.BlockSpec((1, 1), lambda i: (0, 0)),
        ],
        out_shape=[
            jax.ShapeDtypeStruct((nblk, 1, _TOK_BLK), jnp.int32),
            jax.ShapeDtypeStruct((1, 1), jnp.float32),
        ],
    )(zp, embt, am, om)


def _sc_gather(ridx, emb):
    ntok = ridx.shape[0]
    bpw = ntok // _NW
    nch = bpw // _GCHUNK
    mesh = plsc.VectorSubcoreMesh(core_axis_name="c", subcore_axis_name="s")

    @functools.partial(
        pl.kernel,
        mesh=mesh,
        out_type=jax.ShapeDtypeStruct((ntok, _ED), jnp.float32),
        scratch_types=[
            pltpu.VMEM((_GCHUNK,), jnp.int32),
            pltpu.VMEM((_GCHUNK, _ED), jnp.float32),
            pltpu.SemaphoreType.DMA,
        ],
    )
    def gather_k(ridx_hbm, emb_hbm, out_hbm, idx_v, row_v, sem):
        wid = lax.axis_index("s") * _NC + lax.axis_index("c")
        base = wid * bpw

        def chunk(c, carry):
            off = base + c * _GCHUNK
            pltpu.sync_copy(ridx_hbm.at[pl.ds(off, _GCHUNK)], idx_v)
            pltpu.async_copy(emb_hbm.at[idx_v], row_v, sem).wait()
            pltpu.sync_copy(row_v, out_hbm.at[pl.ds(off, _GCHUNK)])
            return carry

        lax.fori_loop(0, nch, chunk, 0)

    return gather_k(ridx, emb)


@functools.lru_cache(maxsize=None)
def _channel_masks(nbatch, npos):
    ntok = nbatch * npos
    cpu = jax.devices("cpu")[0]
    with jax.ensure_compile_time_eval(), jax.default_device(cpu):
        shifts = jnp.arange(_NBIT - 1, -1, -1, dtype=jnp.int32)
        snr_linear = 10.0 ** (_SNR_DB / 10.0)
        noise_std = jnp.sqrt(jnp.asarray(0.5 / snr_linear, dtype=jnp.float32))
        noise = jax.random.normal(jax.random.key(1234), (ntok * _NBIT,),
                                  dtype=jnp.float32) * noise_std
        n = noise.reshape(-1, _NBIT)
        pw = jnp.left_shift(jnp.int32(1), shifts)
        keep1 = (1.0 + n) >= 0.0
        make1 = (-1.0 + n) >= 0.0
        and_mask = jnp.sum(jnp.where(keep1, pw, 0), axis=1).astype(jnp.int32)
        or_mask = jnp.sum(jnp.where(make1, pw, 0), axis=1).astype(jnp.int32)
        import numpy as _np
        nblk = ntok // _TOK_BLK
        am = _np.asarray(and_mask).reshape(nbatch, npos).T.reshape(nblk, 1, _TOK_BLK)
        om = _np.asarray(or_mask).reshape(nbatch, npos).T.reshape(nblk, 1, _TOK_BLK)
        return _np.ascontiguousarray(am), _np.ascontiguousarray(om)


def kernel(z, emb):
    b, c, h, w = z.shape
    hw = h * w
    ntok = b * hw
    zp = jnp.transpose(z.reshape(b, c, hw), (2, 0, 1)).reshape(ntok, _ED)
    embt = jnp.transpose(emb)
    am, om = _channel_masks(b, hw)
    ridx2, loss_sum = _tc_vq(zp, embt, jnp.asarray(am), jnp.asarray(om))
    loss = loss_sum[0, 0] * jnp.float32((1.0 + _BETA) / float(z.size))
    zq = _sc_gather(ridx2.reshape(ntok), emb)
    out = jnp.transpose(zq.reshape(hw, b, _ED), (1, 2, 0)).reshape(b, c, h, w)
    return loss, out


# speedup vs baseline: 5.2854x; 1.1462x over previous
import functools

import jax
import jax.numpy as jnp
from jax import lax
from jax.experimental import pallas as pl
from jax.experimental.pallas import tpu as pltpu
from jax.experimental.pallas import tpu_sc as plsc

_NE = 1024
_ED = 128
_NBIT = 10
_BETA = 0.25
_SNR_DB = 10.0

_TOK_BLK = 2048

_NC, _NS = 2, 16
_NW = _NC * _NS
_GCHUNK = 200


def _vq_argmin_body(zf_ref, embt_ref, am_ref, om_ref, ridx_ref, loss_ref):
    zf = zf_ref[...]
    embt = embt_ref[...]
    mm = jnp.dot(zf, embt, preferred_element_type=jnp.float32)
    zsq = jnp.sum(zf * zf, axis=1, keepdims=True)
    ssq = jnp.sum(embt * embt, axis=0, keepdims=True)
    d = (zsq + ssq) - 2.0 * mm
    dmin = jnp.min(d, axis=1, keepdims=True)
    ids = lax.broadcasted_iota(jnp.int32, (1, _NE), 1)
    idx = jnp.min(jnp.where(d == dmin, ids, _NE), axis=1, keepdims=True)
    idx_l = jnp.transpose(idx, (1, 0))
    ridx_ref[...] = ((idx_l & am_ref[0]) | om_ref[0])[None]

    @pl.when(pl.program_id(0) == 0)
    def _init():
        loss_ref[...] = jnp.zeros_like(loss_ref)

    loss_ref[...] += jnp.sum(dmin, axis=0, keepdims=True)


def _tc_vq(zp, embt, am, om):
    ntok = zp.shape[0]
    nblk = ntok // _TOK_BLK
    return pl.pallas_call(
        _vq_argmin_body,
        grid=(nblk,),
        in_specs=[
            pl.BlockSpec((_TOK_BLK, _ED), lambda i: (i, 0)),
            pl.BlockSpec((_ED, _NE), lambda i: (0, 0)),
            pl.BlockSpec((1, 1, _TOK_BLK), lambda i: (i, 0, 0)),
            pl.BlockSpec((1, 1, _TOK_BLK), lambda i: (i, 0, 0)),
        ],
        out_specs=[
            pl.BlockSpec((1, 1, _TOK_BLK), lambda i: (i, 0, 0)),
            pl.BlockSpec((1, 1), lambda i: (0, 0)),
        ],
        out_shape=[
            jax.ShapeDtypeStruct((nblk, 1, _TOK_BLK), jnp.int32),
            jax.ShapeDtypeStruct((1, 1), jnp.float32),
        ],
    )(zp, embt, am, om)


def _sc_gather(ridx, emb):
    ntok = ridx.shape[0]
    bpw = ntok // _NW
    nch = bpw // _GCHUNK
    mesh = plsc.VectorSubcoreMesh(core_axis_name="c", subcore_axis_name="s")

    @functools.partial(
        pl.kernel,
        mesh=mesh,
        out_type=jax.ShapeDtypeStruct((ntok, _ED), jnp.float32),
        scratch_types=[
            pltpu.VMEM((_GCHUNK,), jnp.int32),
            pltpu.VMEM((_GCHUNK, _ED), jnp.float32),
            pltpu.SemaphoreType.DMA,
        ],
    )
    def gather_k(ridx_hbm, emb_hbm, out_hbm, idx_v, row_v, sem):
        wid = lax.axis_index("s") * _NC + lax.axis_index("c")
        base = wid * bpw

        def chunk(c, carry):
            off = base + c * _GCHUNK
            pltpu.sync_copy(ridx_hbm.at[pl.ds(off, _GCHUNK)], idx_v)
            pltpu.async_copy(emb_hbm.at[idx_v], row_v, sem).wait()
            pltpu.sync_copy(row_v, out_hbm.at[pl.ds(off, _GCHUNK)])
            return carry

        lax.fori_loop(0, nch, chunk, 0)

    return gather_k(ridx, emb)


@functools.lru_cache(maxsize=None)
def _channel_masks(nbatch, npos):
    ntok = nbatch * npos
    cpu = jax.devices("cpu")[0]
    with jax.ensure_compile_time_eval(), jax.default_device(cpu):
        shifts = jnp.arange(_NBIT - 1, -1, -1, dtype=jnp.int32)
        snr_linear = 10.0 ** (_SNR_DB / 10.0)
        noise_std = jnp.sqrt(jnp.asarray(0.5 / snr_linear, dtype=jnp.float32))
        noise = jax.random.normal(jax.random.key(1234), (ntok * _NBIT,),
                                  dtype=jnp.float32) * noise_std
        n = noise.reshape(-1, _NBIT)
        pw = jnp.left_shift(jnp.int32(1), shifts)
        keep1 = (1.0 + n) >= 0.0
        make1 = (-1.0 + n) >= 0.0
        and_mask = jnp.sum(jnp.where(keep1, pw, 0), axis=1).astype(jnp.int32)
        or_mask = jnp.sum(jnp.where(make1, pw, 0), axis=1).astype(jnp.int32)
        import numpy as _np
        nblk = ntok // _TOK_BLK
        am = _np.asarray(and_mask).reshape(nbatch, npos).T.reshape(nblk, 1, _TOK_BLK)
        om = _np.asarray(or_mask).reshape(nbatch, npos).T.reshape(nblk, 1, _TOK_BLK)
        return _np.ascontiguousarray(am), _np.ascontiguousarray(om)


def kernel(z, emb):
    b, c, h, w = z.shape
    hw = h * w
    ntok = b * hw
    zp = jnp.transpose(z.reshape(b, c, hw), (2, 0, 1)).reshape(ntok, _ED)
    embt = jnp.transpose(emb)
    am, om = _channel_masks(b, hw)
    ridx2, loss_sum = _tc_vq(zp, embt, jnp.asarray(am), jnp.asarray(om))
    loss = loss_sum[0, 0] * jnp.float32((1.0 + _BETA) / float(z.size))
    zq = _sc_gather(ridx2.reshape(ntok), emb)
    out = jnp.transpose(zq.reshape(hw, b, _ED), (1, 2, 0)).reshape(b, c, h, w)
    return loss, out


# speedup vs baseline: 5.3068x; 1.0041x over previous
import functools

import jax
import jax.numpy as jnp
from jax import lax
from jax.experimental import pallas as pl
from jax.experimental.pallas import tpu as pltpu
from jax.experimental.pallas import tpu_sc as plsc

_NE = 1024
_ED = 128
_NBIT = 10
_BETA = 0.25
_SNR_DB = 10.0

_TOK_BLK = 2048

_NC, _NS = 2, 16
_NW = _NC * _NS
_GCHUNK = 200


def _vq_argmin_body(zf_ref, embt_ref, am_ref, om_ref, ridx_ref, loss_ref):
    zf = zf_ref[...]
    embt = embt_ref[...]
    mm = jnp.dot(zf, embt, preferred_element_type=jnp.float32)
    zsq = jnp.sum(zf * zf, axis=1, keepdims=True)
    ssq = jnp.sum(embt * embt, axis=0, keepdims=True)
    d = (zsq + ssq) - 2.0 * mm
    dmin = jnp.min(d, axis=1, keepdims=True)
    ids = lax.broadcasted_iota(jnp.int32, (1, _NE), 1)
    idx = jnp.min(jnp.where(d == dmin, ids, _NE), axis=1, keepdims=True)
    idx_l = jnp.transpose(idx, (1, 0))
    ridx_ref[...] = ((idx_l & am_ref[0]) | om_ref[0])[None]

    @pl.when(pl.program_id(0) == 0)
    def _init():
        loss_ref[...] = jnp.zeros_like(loss_ref)

    loss_ref[...] += jnp.sum(dmin, axis=0, keepdims=True)


def _tc_vq(zp, embt, am, om):
    ntok = zp.shape[0]
    nblk = ntok // _TOK_BLK
    return pl.pallas_call(
        _vq_argmin_body,
        grid=(nblk,),
        in_specs=[
            pl.BlockSpec((_TOK_BLK, _ED), lambda i: (i, 0)),
            pl.BlockSpec((_ED, _NE), lambda i: (0, 0)),
            pl.BlockSpec((1, 1, _TOK_BLK), lambda i: (i, 0, 0)),
            pl.BlockSpec((1, 1, _TOK_BLK), lambda i: (i, 0, 0)),
        ],
        out_specs=[
            pl.BlockSpec((1, 1, _TOK_BLK), lambda i: (i, 0, 0)),
            pl.BlockSpec((1, 1), lambda i: (0, 0)),
        ],
        out_shape=[
            jax.ShapeDtypeStruct((nblk, 1, _TOK_BLK), jnp.int32),
            jax.ShapeDtypeStruct((1, 1), jnp.float32),
        ],
    )(zp, embt, am, om)


def _sc_gather(ridx, emb):
    ntok = ridx.shape[0]
    bpw = ntok // _NW
    nch = bpw // _GCHUNK
    mesh = plsc.VectorSubcoreMesh(core_axis_name="c", subcore_axis_name="s")

    @functools.partial(
        pl.kernel,
        mesh=mesh,
        out_type=jax.ShapeDtypeStruct((ntok, _ED), jnp.float32),
        scratch_types=[
            pltpu.VMEM((bpw,), jnp.int32),
            pltpu.VMEM((_GCHUNK, _ED), jnp.float32),
            pltpu.VMEM((_GCHUNK, _ED), jnp.float32),
            pltpu.SemaphoreType.DMA,
            pltpu.SemaphoreType.DMA,
        ],
    )
    def gather_k(ridx_hbm, emb_hbm, out_hbm, idx_v, buf0, buf1, sem0, sem1):
        wid = lax.axis_index("s") * _NC + lax.axis_index("c")
        base = wid * bpw
        pltpu.sync_copy(ridx_hbm.at[pl.ds(base, bpw)], idx_v)

        def chunk2(i, carry):
            o0 = (2 * i) * _GCHUNK
            o1 = o0 + _GCHUNK
            a0 = pltpu.async_copy(emb_hbm.at[idx_v.at[pl.ds(o0, _GCHUNK)]],
                                  buf0, sem0)
            a1 = pltpu.async_copy(emb_hbm.at[idx_v.at[pl.ds(o1, _GCHUNK)]],
                                  buf1, sem1)
            a0.wait()
            pltpu.sync_copy(buf0, out_hbm.at[pl.ds(base + o0, _GCHUNK)])
            a1.wait()
            pltpu.sync_copy(buf1, out_hbm.at[pl.ds(base + o1, _GCHUNK)])
            return carry

        lax.fori_loop(0, nch // 2, chunk2, 0)

    return gather_k(ridx, emb)


@functools.lru_cache(maxsize=None)
def _channel_masks(nbatch, npos):
    ntok = nbatch * npos
    cpu = jax.devices("cpu")[0]
    with jax.ensure_compile_time_eval(), jax.default_device(cpu):
        shifts = jnp.arange(_NBIT - 1, -1, -1, dtype=jnp.int32)
        snr_linear = 10.0 ** (_SNR_DB / 10.0)
        noise_std = jnp.sqrt(jnp.asarray(0.5 / snr_linear, dtype=jnp.float32))
        noise = jax.random.normal(jax.random.key(1234), (ntok * _NBIT,),
                                  dtype=jnp.float32) * noise_std
        n = noise.reshape(-1, _NBIT)
        pw = jnp.left_shift(jnp.int32(1), shifts)
        keep1 = (1.0 + n) >= 0.0
        make1 = (-1.0 + n) >= 0.0
        and_mask = jnp.sum(jnp.where(keep1, pw, 0), axis=1).astype(jnp.int32)
        or_mask = jnp.sum(jnp.where(make1, pw, 0), axis=1).astype(jnp.int32)
        import numpy as _np
        nblk = ntok // _TOK_BLK
        am = _np.asarray(and_mask).reshape(nbatch, npos).T.reshape(nblk, 1, _TOK_BLK)
        om = _np.asarray(or_mask).reshape(nbatch, npos).T.reshape(nblk, 1, _TOK_BLK)
        return _np.ascontiguousarray(am), _np.ascontiguousarray(om)


def kernel(z, emb):
    b, c, h, w = z.shape
    hw = h * w
    ntok = b * hw
    zp = jnp.transpose(z.reshape(b, c, hw), (2, 0, 1)).reshape(ntok, _ED)
    embt = jnp.transpose(emb)
    am, om = _channel_masks(b, hw)
    ridx2, loss_sum = _tc_vq(zp, embt, jnp.asarray(am), jnp.asarray(om))
    loss = loss_sum[0, 0] * jnp.float32((1.0 + _BETA) / float(z.size))
    zq = _sc_gather(ridx2.reshape(ntok), emb)
    out = jnp.transpose(zq.reshape(hw, b, _ED), (1, 2, 0)).reshape(b, c, h, w)
    return loss, out


# speedup vs baseline: 5.4231x; 1.0219x over previous
import functools

import jax
import jax.numpy as jnp
from jax import lax
from jax.experimental import pallas as pl
from jax.experimental.pallas import tpu as pltpu
from jax.experimental.pallas import tpu_sc as plsc

_NE = 1024
_ED = 128
_NBIT = 10
_BETA = 0.25
_SNR_DB = 10.0

_TOK_BLK = 4096

_NC, _NS = 2, 16
_NW = _NC * _NS
_GCHUNK = 200


def _vq_argmin_body(zf_ref, embt_ref, am_ref, om_ref, ridx_ref, loss_ref):
    zf = zf_ref[...]
    embt = embt_ref[...]
    mm = jnp.dot(zf, embt, preferred_element_type=jnp.float32)
    zsq = jnp.sum(zf * zf, axis=1, keepdims=True)
    ssq = jnp.sum(embt * embt, axis=0, keepdims=True)
    d = (zsq + ssq) - 2.0 * mm
    dmin = jnp.min(d, axis=1, keepdims=True)
    ids = lax.broadcasted_iota(jnp.int32, (1, _NE), 1)
    idx = jnp.min(jnp.where(d == dmin, ids, _NE), axis=1, keepdims=True)
    idx_l = jnp.transpose(idx, (1, 0))
    ridx_ref[...] = ((idx_l & am_ref[0]) | om_ref[0])[None]

    @pl.when(pl.program_id(0) == 0)
    def _init():
        loss_ref[...] = jnp.zeros_like(loss_ref)

    loss_ref[...] += jnp.sum(dmin, axis=0, keepdims=True)


def _tc_vq(zp, embt, am, om):
    ntok = zp.shape[0]
    nblk = ntok // _TOK_BLK
    return pl.pallas_call(
        _vq_argmin_body,
        grid=(nblk,),
        in_specs=[
            pl.BlockSpec((_TOK_BLK, _ED), lambda i: (i, 0)),
            pl.BlockSpec((_ED, _NE), lambda i: (0, 0)),
            pl.BlockSpec((1, 1, _TOK_BLK), lambda i: (i, 0, 0)),
            pl.BlockSpec((1, 1, _TOK_BLK), lambda i: (i, 0, 0)),
        ],
        out_specs=[
            pl.BlockSpec((1, 1, _TOK_BLK), lambda i: (i, 0, 0)),
            pl.BlockSpec((1, 1), lambda i: (0, 0)),
        ],
        out_shape=[
            jax.ShapeDtypeStruct((nblk, 1, _TOK_BLK), jnp.int32),
            jax.ShapeDtypeStruct((1, 1), jnp.float32),
        ],
    )(zp, embt, am, om)


def _sc_gather(ridx, emb):
    ntok = ridx.shape[0]
    bpw = ntok // _NW
    nch = bpw // _GCHUNK
    mesh = plsc.VectorSubcoreMesh(core_axis_name="c", subcore_axis_name="s")

    @functools.partial(
        pl.kernel,
        mesh=mesh,
        out_type=jax.ShapeDtypeStruct((ntok, _ED), jnp.float32),
        scratch_types=[
            pltpu.VMEM((bpw,), jnp.int32),
            pltpu.VMEM((_GCHUNK, _ED), jnp.float32),
            pltpu.VMEM((_GCHUNK, _ED), jnp.float32),
            pltpu.SemaphoreType.DMA,
            pltpu.SemaphoreType.DMA,
        ],
    )
    def gather_k(ridx_hbm, emb_hbm, out_hbm, idx_v, buf0, buf1, sem0, sem1):
        wid = lax.axis_index("s") * _NC + lax.axis_index("c")
        base = wid * bpw
        pltpu.sync_copy(ridx_hbm.at[pl.ds(base, bpw)], idx_v)

        def chunk2(i, carry):
            o0 = (2 * i) * _GCHUNK
            o1 = o0 + _GCHUNK
            a0 = pltpu.async_copy(emb_hbm.at[idx_v.at[pl.ds(o0, _GCHUNK)]],
                                  buf0, sem0)
            a1 = pltpu.async_copy(emb_hbm.at[idx_v.at[pl.ds(o1, _GCHUNK)]],
                                  buf1, sem1)
            a0.wait()
            pltpu.sync_copy(buf0, out_hbm.at[pl.ds(base + o0, _GCHUNK)])
            a1.wait()
            pltpu.sync_copy(buf1, out_hbm.at[pl.ds(base + o1, _GCHUNK)])
            return carry

        lax.fori_loop(0, nch // 2, chunk2, 0)

    return gather_k(ridx, emb)


@functools.lru_cache(maxsize=None)
def _channel_masks(nbatch, npos):
    ntok = nbatch * npos
    cpu = jax.devices("cpu")[0]
    with jax.ensure_compile_time_eval(), jax.default_device(cpu):
        shifts = jnp.arange(_NBIT - 1, -1, -1, dtype=jnp.int32)
        snr_linear = 10.0 ** (_SNR_DB / 10.0)
        noise_std = jnp.sqrt(jnp.asarray(0.5 / snr_linear, dtype=jnp.float32))
        noise = jax.random.normal(jax.random.key(1234), (ntok * _NBIT,),
                                  dtype=jnp.float32) * noise_std
        n = noise.reshape(-1, _NBIT)
        pw = jnp.left_shift(jnp.int32(1), shifts)
        keep1 = (1.0 + n) >= 0.0
        make1 = (-1.0 + n) >= 0.0
        and_mask = jnp.sum(jnp.where(keep1, pw, 0), axis=1).astype(jnp.int32)
        or_mask = jnp.sum(jnp.where(make1, pw, 0), axis=1).astype(jnp.int32)
        import numpy as _np
        nblk = ntok // _TOK_BLK
        am = _np.asarray(and_mask).reshape(nbatch, npos).T.reshape(nblk, 1, _TOK_BLK)
        om = _np.asarray(or_mask).reshape(nbatch, npos).T.reshape(nblk, 1, _TOK_BLK)
        return _np.ascontiguousarray(am), _np.ascontiguousarray(om)


def kernel(z, emb):
    b, c, h, w = z.shape
    hw = h * w
    ntok = b * hw
    zp = jnp.transpose(z.reshape(b, c, hw), (2, 0, 1)).reshape(ntok, _ED)
    embt = jnp.transpose(emb)
    am, om = _channel_masks(b, hw)
    ridx2, loss_sum = _tc_vq(zp, embt, jnp.asarray(am), jnp.asarray(om))
    loss = loss_sum[0, 0] * jnp.float32((1.0 + _BETA) / float(z.size))
    zq = _sc_gather(ridx2.reshape(ntok), emb)
    out = jnp.transpose(zq.reshape(hw, b, _ED), (1, 2, 0)).reshape(b, c, h, w)
    return loss, out
